# baseline (speedup 1.0000x reference)
import functools

import jax
import jax.numpy as jnp
from jax import lax
from jax.experimental import pallas as pl
from jax.experimental.pallas import tpu as pltpu
from jax.experimental.pallas import tpu_sc as plsc

B = 4096
N_ITEMS = 200
D = 64
T = 50
BT = B * T
NC = 2
NS = 16
NW = NC * NS
PER_W = BT // NW
CH = 128
NCH = PER_W // CH
L = 16


def _body(table, ids, out, idx_v, buf, gsem):
    wid = lax.axis_index("s") * NC + lax.axis_index("c")
    g0 = wid * PER_W

    pltpu.sync_copy(ids.at[pl.ds(g0, PER_W)], idx_v)

    lane = lax.iota(jnp.int32, L)

    def off_body(i, carry):
        g = (g0 + i * L) + lane
        sl = pl.ds(i * L, L)
        idx_v[sl] = idx_v[sl] + lax.div(g, T) * N_ITEMS
        return carry

    lax.fori_loop(0, PER_W // L, off_body, 0)

    def ch_body(c, carry):
        isl = idx_v.at[pl.ds(c * CH, CH)]
        pltpu.async_copy(table.at[isl], buf, gsem).wait()
        pltpu.sync_copy(buf, out.at[pl.ds(g0 + c * CH, CH)])
        return carry

    lax.fori_loop(0, NCH, ch_body, 0)


def kernel(all_embeddings, target_ids):
    table = all_embeddings.reshape(B * N_ITEMS, D)
    ids = target_ids.astype(jnp.int32).reshape(BT)
    mesh = plsc.VectorSubcoreMesh(core_axis_name="c", subcore_axis_name="s")
    run = pl.kernel(
        _body,
        mesh=mesh,
        out_type=jax.ShapeDtypeStruct((BT, D), jnp.float32),
        scratch_types=[
            pltpu.VMEM((PER_W,), jnp.int32),
            pltpu.VMEM((CH, D), jnp.float32),
            pltpu.SemaphoreType.DMA,
        ],
        compiler_params=pltpu.CompilerParams(use_tc_tiling_on_sc=False),
    )
    return run(table, ids).reshape(B, T, D)


# speedup vs baseline: 1.1844x; 1.1844x over previous
import jax
import jax.numpy as jnp
from jax import lax
from jax.experimental import pallas as pl
from jax.experimental.pallas import tpu as pltpu
from jax.experimental.pallas import tpu_sc as plsc

B = 4096
N_ITEMS = 200
D = 64
T = 50
NC = 2
NS = 16
NW = NC * NS
BPW = B // NW
GRP = 4
IDG = (T + 15) // 16


def _body(table, ids, out, idx_v, sa, sb, oa, ob, gs0, gs1, os0, os1):
    wid = lax.axis_index("s") * NC + lax.axis_index("c")
    b0 = wid * BPW

    slabs = [sa, sb]
    outgs = [oa, ob]
    gsems = [gs0, gs1]
    osems = [os0, os1]

    pltpu.sync_copy(ids.at[pl.ds(b0, BPW)], idx_v)

    pltpu.async_copy(table.at[b0], sa, gs0)

    def select_rows(k, slab_b, outg_b, slot):
        def g_body(g, carry):
            o = lax.min(g * 16, T - 16)
            tv = idx_v[k, pl.ds(o, 16)]
            for i in range(16):
                sid = tv[i]
                for q in range(D // 16):
                    cs = pl.ds(q * 16, 16)
                    outg_b[slot, o + i, cs] = slab_b[sid, cs]
            return carry

        lax.fori_loop(0, IDG, g_body, 0)

    def g8_body(g8, carry):
        for j in range(8):
            k = g8 * 8 + j
            cur = j % 2
            obuf = j // 4

            @pl.when(k + 1 < BPW)
            def _():
                pltpu.async_copy(table.at[b0 + k + 1], slabs[1 - cur],
                                 gsems[1 - cur])

            if j % 4 == 0:
                @pl.when(k >= 2 * GRP)
                def _():
                    pltpu.make_async_copy(outgs[obuf],
                                          out.at[pl.ds(0, GRP)],
                                          osems[obuf]).wait()

            pltpu.make_async_copy(table.at[b0], slabs[cur], gsems[cur]).wait()
            select_rows(k, slabs[cur], outgs[obuf], j % 4)

            if j % 4 == 3:
                grp0 = k - 3
                pltpu.async_copy(outgs[obuf],
                                 out.at[pl.ds(b0 + grp0, GRP)],
                                 osems[obuf])
        return carry

    lax.fori_loop(0, BPW // 8, g8_body, 0)

    pltpu.make_async_copy(oa, out.at[pl.ds(0, GRP)], os0).wait()
    pltpu.make_async_copy(ob, out.at[pl.ds(0, GRP)], os1).wait()


def kernel(all_embeddings, target_ids):
    ids = target_ids.astype(jnp.int32)
    mesh = plsc.VectorSubcoreMesh(core_axis_name="c", subcore_axis_name="s")
    run = pl.kernel(
        _body,
        mesh=mesh,
        out_type=jax.ShapeDtypeStruct((B, T, D), jnp.float32),
        scratch_types=[
            pltpu.VMEM((BPW, T), jnp.int32),
            pltpu.VMEM((N_ITEMS, D), jnp.float32),
            pltpu.VMEM((N_ITEMS, D), jnp.float32),
            pltpu.VMEM((GRP, T, D), jnp.float32),
            pltpu.VMEM((GRP, T, D), jnp.float32),
            pltpu.SemaphoreType.DMA,
            pltpu.SemaphoreType.DMA,
            pltpu.SemaphoreType.DMA,
            pltpu.SemaphoreType.DMA,
        ],
    )
    return run(all_embeddings, ids)
